# Initial kernel scaffold; baseline (speedup 1.0000x reference)
import jax
import jax.numpy as jnp
from jax.experimental import pallas as pl


def kernel(ids, sym, emb_table, fc_w, fc_b):
    raise NotImplementedError("write your pallas kernel here")


# speedup vs baseline: 5.1274x; 5.1274x over previous
import functools

import jax
import jax.numpy as jnp
from jax import lax
from jax.experimental import pallas as pl
from jax.experimental.pallas import tpu as pltpu
from jax.experimental.pallas import tpu_sc as plsc

_B = 4096
_L = 50
_LP = 64
_H = 128
_SYM = 32
_C = 16

_NC = 2
_NS = 16
_NW = _NC * _NS
_BPW = _B // _NW
_LANES = 16


def _avg_pool_sc(ids_pad, emb_table):
    mesh = plsc.VectorSubcoreMesh(core_axis_name="c", subcore_axis_name="s")

    @functools.partial(
        pl.kernel,
        out_type=jax.ShapeDtypeStruct((_B, _H), jnp.float32),
        mesh=mesh,
        scratch_types=[
            pltpu.VMEM((_BPW, _LP), jnp.int32),
            pltpu.VMEM((2, _L, _H), jnp.float32),
            pltpu.VMEM((_BPW, _H), jnp.float32),
            pltpu.SemaphoreType.DMA,
            pltpu.SemaphoreType.DMA,
        ],
    )
    def body(ids_hbm, table_hbm, avg_hbm, ids_v, rows_v, avg_v, sem0, sem1):
        wid = lax.axis_index("s") * _NC + lax.axis_index("c")
        base = wid * _BPW
        pltpu.sync_copy(ids_hbm.at[pl.ds(base, _BPW)], ids_v)

        sems = (sem0, sem1)

        def issue(row, buf):
            return pltpu.async_copy(
                table_hbm.at[ids_v.at[row, pl.ds(0, _L)]],
                rows_v.at[buf],
                sems[buf],
            )

        def drain(row, buf):
            pltpu.make_async_copy(
                table_hbm.at[ids_v.at[row, pl.ds(0, _L)]],
                rows_v.at[buf],
                sems[buf],
            ).wait()

        def consume(row, buf):
            for k in range(_H // _LANES):
                acc = rows_v[buf, 0, pl.ds(k * _LANES, _LANES)]
                for l in range(1, _L):
                    acc = acc + rows_v[buf, l, pl.ds(k * _LANES, _LANES)]
                avg_v[row, pl.ds(k * _LANES, _LANES)] = acc

        issue(0, 0)

        def loop_body(r2, carry):
            issue(jnp.minimum(r2 + 1, _BPW - 1), 1)
            drain(r2, 0)
            consume(r2, 0)
            issue(jnp.minimum(r2 + 2, _BPW - 1), 0)
            drain(jnp.minimum(r2 + 1, _BPW - 1), 1)
            consume(r2 + 1, 1)
            return carry

        lax.fori_loop(0, _BPW // 2, lambda i, c: loop_body(i * 2, c), 0)
        drain(_BPW - 1, 0)

        pltpu.sync_copy(avg_v, avg_hbm.at[pl.ds(base, _BPW)])

    return body(ids_pad, emb_table)


def _fc_body(emb_sum_ref, ids_ref, sym_ref, w1_ref, w2_ref, b_ref, out_ref):
    cnt = jnp.sum(jnp.where(ids_ref[...] != 0, 1.0, 0.0), axis=1, keepdims=True)
    avg = emb_sum_ref[...] * (1.0 / jnp.maximum(cnt, 1.0))
    out_ref[...] = (
        jnp.dot(avg, w1_ref[...], preferred_element_type=jnp.float32)
        + jnp.dot(sym_ref[...], w2_ref[...], preferred_element_type=jnp.float32)
        + b_ref[...]
    )


def kernel(ids, sym, emb_table, fc_w, fc_b):
    ids = ids.astype(jnp.int32)
    ids_pad = jnp.pad(ids, ((0, 0), (0, _LP - _L)))
    emb_sum = _avg_pool_sc(ids_pad, emb_table)

    w1 = fc_w[:, :_H].T
    w2 = fc_w[:, _H:].T
    out = pl.pallas_call(
        _fc_body,
        out_shape=jax.ShapeDtypeStruct((_B, _C), jnp.float32),
    )(emb_sum, ids, sym, w1, w2, fc_b.reshape(1, _C))
    return out
